# baseline (device time: 335160 ns/iter reference)
import jax
import jax.numpy as jnp
from jax import lax
from jax.experimental import pallas as pl
from jax.experimental.pallas import tpu as pltpu

N_DEV = 4
K_T = 16
N_SUB = 4
_GELU_C = 0.7978845608028654


def kernel(x, w_mat):
    m_per, k = x.shape
    n = w_mat.shape[1]
    n_per = n // N_DEV
    n_t = n_per // N_SUB
    k_t = k // K_T

    my = lax.axis_index("i")
    perm = (my + 1 + jnp.arange(N_DEV, dtype=jnp.int32)) % N_DEV

    def body(perm_ref, x_win, w_ref, out_ref, x_bf, acc, y_send, own_buf,
             comm, send_sems, recv_sems, copy_sems, own_sem):
        j = pl.program_id(0)
        nh = pl.program_id(1)
        kk = pl.program_id(2)
        me = lax.axis_index("i")
        tgt = perm_ref[j]
        si = j * N_SUB + nh
        slot = lax.rem(si, 3)

        @pl.when(jnp.logical_and(j == 0, nh == 0))
        def _load_x():
            x_bf[:, pl.ds(kk * k_t, k_t)] = x_win[...].astype(jnp.bfloat16)

        xa = x_bf[:, pl.ds(kk * k_t, k_t)]
        part = jnp.dot(xa, w_ref[...].astype(jnp.bfloat16),
                       preferred_element_type=jnp.float32)

        @pl.when(kk == 0)
        def _init():
            acc[...] = part

        @pl.when(kk > 0)
        def _accum():
            acc[...] = acc[...] + part

        @pl.when(kk == K_T - 1)
        def _sub_block_done():
            @pl.when(jnp.logical_and(si >= 3, j < N_DEV - 1))
            def _():
                pltpu.make_async_remote_copy(
                    src_ref=y_send.at[slot],
                    dst_ref=comm.at[0, 0],
                    send_sem=send_sems.at[slot],
                    recv_sem=recv_sems.at[me, nh],
                    device_id=(me,),
                    device_id_type=pl.DeviceIdType.MESH,
                ).wait_send()

            @pl.when(jnp.logical_and(j == N_DEV - 1, nh > 0))
            def _():
                pltpu.make_async_copy(
                    own_buf,
                    out_ref.at[pl.ds(me * m_per, m_per), pl.ds(0, n_t)],
                    own_sem,
                ).wait()

            n_chunks = 4
            mc = m_per // n_chunks
            for c in range(n_chunks):
                a = acc[pl.ds(c * mc, mc), :]
                g = 0.5 * a * (1.0 + jnp.tanh(_GELU_C * (a + 0.044715 * a * a * a)))
                g = g.astype(jnp.bfloat16)

                @pl.when(j < N_DEV - 1)
                def _(g=g, c=c):
                    y_send[slot, pl.ds(c * mc, mc), :] = g

                @pl.when(j == N_DEV - 1)
                def _(g=g, c=c):
                    own_buf[pl.ds(c * mc, mc), :] = g

            @pl.when(j < N_DEV - 1)
            def _send():
                s_off = lax.rem(me - tgt + N_DEV, N_DEV) - 1
                pltpu.make_async_remote_copy(
                    src_ref=y_send.at[slot],
                    dst_ref=comm.at[s_off, nh],
                    send_sem=send_sems.at[slot],
                    recv_sem=recv_sems.at[me, nh],
                    device_id=(tgt,),
                    device_id_type=pl.DeviceIdType.MESH,
                ).start()

            @pl.when(j == N_DEV - 1)
            def _own():
                pltpu.make_async_copy(
                    own_buf,
                    out_ref.at[pl.ds(me * m_per, m_per),
                               pl.ds(nh * n_t, n_t)],
                    own_sem,
                ).start()

            @pl.when(jnp.logical_and(j == N_DEV - 1, nh == N_SUB - 1))
            def _finish():
                for step in range(N_DEV - 1):
                    src = (me - 1 - step) % N_DEV
                    s_off = lax.rem(src - me + N_DEV, N_DEV) - 1
                    for q in range(N_SUB):
                        pltpu.make_async_remote_copy(
                            src_ref=y_send.at[0],
                            dst_ref=comm.at[s_off, q],
                            send_sem=send_sems.at[0],
                            recv_sem=recv_sems.at[src, q],
                            device_id=(me,),
                            device_id_type=pl.DeviceIdType.MESH,
                        ).wait_recv()
                        pltpu.make_async_copy(
                            comm.at[s_off, q],
                            out_ref.at[pl.ds(src * m_per, m_per),
                                       pl.ds(q * n_t, n_t)],
                            copy_sems.at[s_off, q],
                        ).start()
                for step in range(N_DEV - 1):
                    src = (me - 1 - step) % N_DEV
                    s_off = lax.rem(src - me + N_DEV, N_DEV) - 1
                    for q in range(N_SUB):
                        pltpu.make_async_copy(
                            comm.at[s_off, q],
                            out_ref.at[pl.ds(src * m_per, m_per),
                                       pl.ds(q * n_t, n_t)],
                            copy_sems.at[s_off, q],
                        ).wait()
                for s in range(3):
                    pltpu.make_async_remote_copy(
                        src_ref=y_send.at[s],
                        dst_ref=comm.at[0, 0],
                        send_sem=send_sems.at[s],
                        recv_sem=recv_sems.at[me, 0],
                        device_id=(me,),
                        device_id_type=pl.DeviceIdType.MESH,
                    ).wait_send()
                pltpu.make_async_copy(
                    own_buf,
                    out_ref.at[pl.ds(me * m_per, m_per), pl.ds(0, n_t)],
                    own_sem,
                ).wait()

    grid_spec = pltpu.PrefetchScalarGridSpec(
        num_scalar_prefetch=1,
        grid=(N_DEV, N_SUB, K_T),
        in_specs=[
            pl.BlockSpec(
                (m_per, k_t),
                lambda j, nh, kk, perm: (
                    0,
                    jnp.where(jnp.logical_and(j == 0, nh == 0), kk, K_T - 1),
                ),
            ),
            pl.BlockSpec(
                (k_t, n_t),
                lambda j, nh, kk, perm: (kk, perm[j] * N_SUB + nh),
            ),
        ],
        out_specs=pl.BlockSpec(memory_space=pltpu.MemorySpace.HBM),
        scratch_shapes=[
            pltpu.VMEM((m_per, k), jnp.bfloat16),
            pltpu.VMEM((m_per, n_t), jnp.float32),
            pltpu.VMEM((3, m_per, n_t), jnp.bfloat16),
            pltpu.VMEM((m_per, n_t), jnp.bfloat16),
            pltpu.VMEM((N_DEV - 1, N_SUB, m_per, n_t), jnp.bfloat16),
            pltpu.SemaphoreType.DMA((3,)),
            pltpu.SemaphoreType.DMA((N_DEV, N_SUB)),
            pltpu.SemaphoreType.DMA((N_DEV - 1, N_SUB)),
            pltpu.SemaphoreType.DMA,
        ],
    )
    return pl.pallas_call(
        body,
        grid_spec=grid_spec,
        out_shape=jax.ShapeDtypeStruct((N_DEV * m_per, n_per), jnp.bfloat16),
        compiler_params=pltpu.CompilerParams(
            dimension_semantics=("arbitrary", "arbitrary", "arbitrary"),
            vmem_limit_bytes=63 * 1024 * 1024,
        ),
    )(perm, x, w_mat)


# device time: 261795 ns/iter; 1.2802x vs baseline; 1.2802x over previous
import jax
import jax.numpy as jnp
from jax import lax
from jax.experimental import pallas as pl
from jax.experimental.pallas import tpu as pltpu

N_DEV = 4
K_T = 16
_GELU_C = 0.7978845608028654


def kernel(x, w_mat):
    m_per, k = x.shape
    n = w_mat.shape[1]
    n_per = n // N_DEV
    k_t = k // K_T
    lt = k_t // 2

    my = lax.axis_index("i")
    perm = (my + 1 + jnp.arange(N_DEV, dtype=jnp.int32)) % N_DEV

    def body(perm_ref, x_ref, w_ref, out_ref, x_bf, xtmp, acc, y_send,
             load_sems, send_sems, recv_sems, copy_sem):
        j = pl.program_id(0)
        kk = pl.program_id(1)
        me = lax.axis_index("i")

        @pl.when(j == 0)
        def _stream_x():
            @pl.when(kk == 0)
            def _prologue():
                for i in range(2):
                    pltpu.make_async_copy(
                        x_ref.at[:, pl.ds(i * lt, lt)],
                        xtmp.at[i],
                        load_sems.at[i],
                    ).start()

            for i in range(2):
                t = kk * 2 + i
                pltpu.make_async_copy(
                    x_ref.at[:, pl.ds(t * lt, lt)],
                    xtmp.at[i],
                    load_sems.at[i],
                ).wait()
                x_bf[:, pl.ds(t * lt, lt)] = xtmp[i].astype(jnp.bfloat16)

            @pl.when(kk < K_T - 1)
            def _prefetch():
                for i in range(2):
                    t = (kk + 1) * 2 + i
                    pltpu.make_async_copy(
                        x_ref.at[:, pl.ds(t * lt, lt)],
                        xtmp.at[i],
                        load_sems.at[i],
                    ).start()

        wb = w_ref[...].astype(jnp.bfloat16)
        for h in range(2):
            rows = pl.ds(h * (m_per // 2), m_per // 2)
            xa = x_bf[rows, pl.ds(kk * k_t, k_t)]
            part = jnp.dot(xa, wb, preferred_element_type=jnp.float32)

            @pl.when(kk == 0)
            def _init(part=part, rows=rows):
                acc[rows, :] = part

            @pl.when(kk > 0)
            def _accum(part=part, rows=rows):
                acc[rows, :] = acc[rows, :] + part

        @pl.when(kk == K_T - 1)
        def _block_done():
            slot = lax.rem(j, 2)

            @pl.when(j >= 2)
            def _():
                pltpu.make_async_remote_copy(
                    src_ref=y_send.at[slot],
                    dst_ref=out_ref.at[pl.ds(me * m_per, m_per), :],
                    send_sem=send_sems.at[slot],
                    recv_sem=recv_sems.at[me],
                    device_id=(me,),
                    device_id_type=pl.DeviceIdType.MESH,
                ).wait_send()

            n_chunks = 8
            mc = m_per // n_chunks
            for c in range(n_chunks):
                a = acc[pl.ds(c * mc, mc), :]
                yc = 0.5 * a * (1.0 + jnp.tanh(_GELU_C * (a + 0.044715 * a * a * a)))
                y_send[slot, pl.ds(c * mc, mc), :] = yc.astype(jnp.bfloat16)

            @pl.when(j < N_DEV - 1)
            def _send():
                pltpu.make_async_remote_copy(
                    src_ref=y_send.at[slot],
                    dst_ref=out_ref.at[pl.ds(me * m_per, m_per), :],
                    send_sem=send_sems.at[slot],
                    recv_sem=recv_sems.at[me],
                    device_id=(perm_ref[j],),
                    device_id_type=pl.DeviceIdType.MESH,
                ).start()

            @pl.when(j == N_DEV - 1)
            def _finish():
                local = pltpu.make_async_copy(
                    y_send.at[slot],
                    out_ref.at[pl.ds(me * m_per, m_per), :],
                    copy_sem,
                )
                local.start()
                for step in range(N_DEV - 1):
                    src = (me - 1 - step) % N_DEV
                    pltpu.make_async_remote_copy(
                        src_ref=y_send.at[0],
                        dst_ref=out_ref.at[pl.ds(src * m_per, m_per), :],
                        send_sem=send_sems.at[0],
                        recv_sem=recv_sems.at[src],
                        device_id=(me,),
                        device_id_type=pl.DeviceIdType.MESH,
                    ).wait_recv()
                pltpu.make_async_remote_copy(
                    src_ref=y_send.at[0],
                    dst_ref=out_ref.at[pl.ds(me * m_per, m_per), :],
                    send_sem=send_sems.at[0],
                    recv_sem=recv_sems.at[me],
                    device_id=(me,),
                    device_id_type=pl.DeviceIdType.MESH,
                ).wait_send()
                local.wait()

    grid_spec = pltpu.PrefetchScalarGridSpec(
        num_scalar_prefetch=1,
        grid=(N_DEV, K_T),
        in_specs=[
            pl.BlockSpec(memory_space=pltpu.MemorySpace.HBM),
            pl.BlockSpec((k_t, n_per), lambda j, kk, perm: (kk, perm[j])),
        ],
        out_specs=pl.BlockSpec(memory_space=pltpu.MemorySpace.HBM),
        scratch_shapes=[
            pltpu.VMEM((m_per, k), jnp.bfloat16),
            pltpu.VMEM((2, m_per, lt), jnp.float32),
            pltpu.VMEM((m_per, n_per), jnp.float32),
            pltpu.VMEM((2, m_per, n_per), jnp.bfloat16),
            pltpu.SemaphoreType.DMA((2,)),
            pltpu.SemaphoreType.DMA((2,)),
            pltpu.SemaphoreType.DMA((N_DEV,)),
            pltpu.SemaphoreType.DMA,
        ],
    )
    return pl.pallas_call(
        body,
        grid_spec=grid_spec,
        out_shape=jax.ShapeDtypeStruct((N_DEV * m_per, n_per), jnp.bfloat16),
        compiler_params=pltpu.CompilerParams(
            dimension_semantics=("arbitrary", "arbitrary"),
            vmem_limit_bytes=63 * 1024 * 1024,
        ),
    )(perm, x, w_mat)


# device time: 217447 ns/iter; 1.5413x vs baseline; 1.2039x over previous
import jax
import jax.numpy as jnp
from jax import lax
from jax.experimental import pallas as pl
from jax.experimental.pallas import tpu as pltpu

N_DEV = 4
K_T = 8
_GELU_C = 0.7978845608028654


def kernel(x, w_mat):
    m_per, k = x.shape
    n = w_mat.shape[1]
    n_per = n // N_DEV
    k_t = k // K_T

    my = lax.axis_index("i")
    perm = (my + 1 + jnp.arange(N_DEV, dtype=jnp.int32)) % N_DEV

    def body(perm_ref, x_ref, w_ref, out_ref, acc, y_send,
             send_sems, recv_sems, copy_sem):
        j = pl.program_id(0)
        kk = pl.program_id(1)
        me = lax.axis_index("i")

        wb = w_ref[...].astype(jnp.bfloat16)
        for h in range(2):
            rows = pl.ds(h * (m_per // 2), m_per // 2)
            xa = x_ref[rows, :].astype(jnp.bfloat16)
            part = jnp.dot(xa, wb, preferred_element_type=jnp.float32)

            @pl.when(kk == 0)
            def _init(part=part, rows=rows):
                acc[rows, :] = part

            @pl.when(kk > 0)
            def _accum(part=part, rows=rows):
                acc[rows, :] = acc[rows, :] + part

        @pl.when(kk == K_T - 1)
        def _block_done():
            slot = lax.rem(j, 2)

            @pl.when(j >= 2)
            def _():
                pltpu.make_async_remote_copy(
                    src_ref=y_send.at[slot],
                    dst_ref=out_ref.at[pl.ds(me * m_per, m_per), :],
                    send_sem=send_sems.at[slot],
                    recv_sem=recv_sems.at[me],
                    device_id=(me,),
                    device_id_type=pl.DeviceIdType.MESH,
                ).wait_send()

            n_chunks = 8
            mc = m_per // n_chunks
            for c in range(n_chunks):
                a = acc[pl.ds(c * mc, mc), :]
                yc = 0.5 * a * (1.0 + jnp.tanh(_GELU_C * (a + 0.044715 * a * a * a)))
                y_send[slot, pl.ds(c * mc, mc), :] = yc.astype(jnp.bfloat16)

            @pl.when(j < N_DEV - 1)
            def _send():
                pltpu.make_async_remote_copy(
                    src_ref=y_send.at[slot],
                    dst_ref=out_ref.at[pl.ds(me * m_per, m_per), :],
                    send_sem=send_sems.at[slot],
                    recv_sem=recv_sems.at[me],
                    device_id=(perm_ref[j],),
                    device_id_type=pl.DeviceIdType.MESH,
                ).start()

            @pl.when(j == N_DEV - 1)
            def _finish():
                local = pltpu.make_async_copy(
                    y_send.at[slot],
                    out_ref.at[pl.ds(me * m_per, m_per), :],
                    copy_sem,
                )
                local.start()
                for step in range(N_DEV - 1):
                    src = (me - 1 - step) % N_DEV
                    pltpu.make_async_remote_copy(
                        src_ref=y_send.at[0],
                        dst_ref=out_ref.at[pl.ds(src * m_per, m_per), :],
                        send_sem=send_sems.at[0],
                        recv_sem=recv_sems.at[src],
                        device_id=(me,),
                        device_id_type=pl.DeviceIdType.MESH,
                    ).wait_recv()
                pltpu.make_async_remote_copy(
                    src_ref=y_send.at[0],
                    dst_ref=out_ref.at[pl.ds(me * m_per, m_per), :],
                    send_sem=send_sems.at[0],
                    recv_sem=recv_sems.at[me],
                    device_id=(me,),
                    device_id_type=pl.DeviceIdType.MESH,
                ).wait_send()
                local.wait()

    grid_spec = pltpu.PrefetchScalarGridSpec(
        num_scalar_prefetch=1,
        grid=(N_DEV, K_T),
        in_specs=[
            pl.BlockSpec((m_per, k_t), lambda j, kk, perm: (0, kk)),
            pl.BlockSpec((k_t, n_per), lambda j, kk, perm: (kk, perm[j])),
        ],
        out_specs=pl.BlockSpec(memory_space=pltpu.MemorySpace.HBM),
        scratch_shapes=[
            pltpu.VMEM((m_per, n_per), jnp.float32),
            pltpu.VMEM((2, m_per, n_per), jnp.bfloat16),
            pltpu.SemaphoreType.DMA((2,)),
            pltpu.SemaphoreType.DMA((N_DEV,)),
            pltpu.SemaphoreType.DMA,
        ],
    )
    return pl.pallas_call(
        body,
        grid_spec=grid_spec,
        out_shape=jax.ShapeDtypeStruct((N_DEV * m_per, n_per), jnp.bfloat16),
        compiler_params=pltpu.CompilerParams(
            dimension_semantics=("arbitrary", "arbitrary"),
            vmem_limit_bytes=63 * 1024 * 1024,
        ),
    )(perm, x, w_mat)


# device time: 213292 ns/iter; 1.5714x vs baseline; 1.0195x over previous
import jax
import jax.numpy as jnp
from jax import lax
from jax.experimental import pallas as pl
from jax.experimental.pallas import tpu as pltpu

N_DEV = 4
K_T = 8
_GELU_C = 0.7978845608028654


def kernel(x, w_mat):
    m_per, k = x.shape
    n = w_mat.shape[1]
    n_per = n // N_DEV
    k_t = k // K_T

    my = lax.axis_index("i")
    perm = (my + 1 + jnp.arange(N_DEV, dtype=jnp.int32)) % N_DEV

    def body(perm_ref, x_ref, w_ref, out_ref, acc, y_send,
             send_sems, recv_sems, copy_sem):
        j = pl.program_id(0)
        kk = pl.program_id(1)
        me = lax.axis_index("i")

        wb = w_ref[...].astype(jnp.bfloat16)
        xa = x_ref[...].astype(jnp.bfloat16)
        part = jnp.dot(xa, wb, preferred_element_type=jnp.float32)

        @pl.when(kk == 0)
        def _init():
            acc[...] = part

        @pl.when(kk > 0)
        def _accum():
            acc[...] = acc[...] + part

        @pl.when(kk == K_T - 1)
        def _block_done():
            slot = lax.rem(j, 2)

            @pl.when(j >= 2)
            def _():
                pltpu.make_async_remote_copy(
                    src_ref=y_send.at[slot],
                    dst_ref=out_ref.at[pl.ds(me * m_per, m_per), :],
                    send_sem=send_sems.at[slot],
                    recv_sem=recv_sems.at[me],
                    device_id=(me,),
                    device_id_type=pl.DeviceIdType.MESH,
                ).wait_send()

            n_chunks = 8
            mc = m_per // n_chunks
            for c in range(n_chunks):
                a = acc[pl.ds(c * mc, mc), :]
                yc = 0.5 * a * (1.0 + jnp.tanh(_GELU_C * (a + 0.044715 * a * a * a)))
                y_send[slot, pl.ds(c * mc, mc), :] = yc.astype(jnp.bfloat16)

            @pl.when(j < N_DEV - 1)
            def _send():
                pltpu.make_async_remote_copy(
                    src_ref=y_send.at[slot],
                    dst_ref=out_ref.at[pl.ds(me * m_per, m_per), :],
                    send_sem=send_sems.at[slot],
                    recv_sem=recv_sems.at[me],
                    device_id=(perm_ref[j],),
                    device_id_type=pl.DeviceIdType.MESH,
                ).start()

            @pl.when(j == N_DEV - 1)
            def _finish():
                local = pltpu.make_async_copy(
                    y_send.at[slot],
                    out_ref.at[pl.ds(me * m_per, m_per), :],
                    copy_sem,
                )
                local.start()
                for step in range(N_DEV - 1):
                    src = (me - 1 - step) % N_DEV
                    pltpu.make_async_remote_copy(
                        src_ref=y_send.at[0],
                        dst_ref=out_ref.at[pl.ds(src * m_per, m_per), :],
                        send_sem=send_sems.at[0],
                        recv_sem=recv_sems.at[src],
                        device_id=(me,),
                        device_id_type=pl.DeviceIdType.MESH,
                    ).wait_recv()
                pltpu.make_async_remote_copy(
                    src_ref=y_send.at[0],
                    dst_ref=out_ref.at[pl.ds(me * m_per, m_per), :],
                    send_sem=send_sems.at[0],
                    recv_sem=recv_sems.at[me],
                    device_id=(me,),
                    device_id_type=pl.DeviceIdType.MESH,
                ).wait_send()
                local.wait()

    grid_spec = pltpu.PrefetchScalarGridSpec(
        num_scalar_prefetch=1,
        grid=(N_DEV, K_T),
        in_specs=[
            pl.BlockSpec((m_per, k_t), lambda j, kk, perm: (0, kk)),
            pl.BlockSpec((k_t, n_per), lambda j, kk, perm: (kk, perm[j])),
        ],
        out_specs=pl.BlockSpec(memory_space=pltpu.MemorySpace.HBM),
        scratch_shapes=[
            pltpu.VMEM((m_per, n_per), jnp.float32),
            pltpu.VMEM((2, m_per, n_per), jnp.bfloat16),
            pltpu.SemaphoreType.DMA((2,)),
            pltpu.SemaphoreType.DMA((N_DEV,)),
            pltpu.SemaphoreType.DMA,
        ],
    )
    return pl.pallas_call(
        body,
        grid_spec=grid_spec,
        out_shape=jax.ShapeDtypeStruct((N_DEV * m_per, n_per), jnp.bfloat16),
        compiler_params=pltpu.CompilerParams(
            dimension_semantics=("arbitrary", "arbitrary"),
            vmem_limit_bytes=63 * 1024 * 1024,
        ),
    )(perm, x, w_mat)
